# baseline (speedup 1.0000x reference)
import functools

import jax
import jax.numpy as jnp
from jax import lax
from jax.experimental import pallas as pl
from jax.experimental.pallas import tpu as pltpu
from jax.experimental.pallas import tpu_sc as plsc

_NC = 2
_NS = 16
_B = 80


def _att_body(ent_ref, rel_ref, out_ref):
    z = jnp.dot(ent_ref[...], rel_ref[...].T, preferred_element_type=jnp.float32)
    z = z - jnp.max(z, axis=1, keepdims=True)
    e = jnp.exp(z)
    out_ref[...] = e / jnp.sum(e, axis=1, keepdims=True)


def _entity_relation_att(entity_emb, relation_emb):
    n_ent, _ = entity_emb.shape
    n_rel = relation_emb.shape[0]
    return pl.pallas_call(
        _att_body,
        out_shape=jax.ShapeDtypeStruct((n_ent, n_rel), jnp.float32),
    )(entity_emb, relation_emb)


def _edge_body(att_ref, ent_ref, rel_ref, head_ref, tail_ref, type_ref,
               imp_ref, out_ref,
               hb0, tb0, yb0, ib0, hb1, tb1, yb1, ib1,
               abuf, aidx, rows, relrows, zrow, acc,
               sem_m0, sem_m1, sem_g):
    c = lax.axis_index("c")
    s = lax.axis_index("s")
    n_ent, d = acc.shape
    n_rel = rel_ref.shape[0]
    e_total = head_ref.shape[0]
    e_per_w = e_total // (_NC * _NS)
    n_chunks = e_per_w // _B
    rows_per_sub = 624
    n_tail = n_ent - rows_per_sub * _NS
    rb = s * rows_per_sub
    wid = c * _NS + s
    eb = wid * e_per_w
    iota16 = lax.iota(jnp.int32, 16)
    z16 = jnp.zeros((16,), jnp.float32)

    nz = zrow.shape[0]

    @plsc.parallel_loop(0, nz)
    def _zero_rows(i):
        for k in range(d // 16):
            zrow[i, pl.ds(k * 16, 16)] = z16

    @plsc.parallel_loop(0, rows_per_sub // nz)
    def _zero_acc(j):
        pltpu.sync_copy(zrow, acc.at[pl.ds(rb + j * nz, nz)])

    @pl.when(s == 0)
    def _zero_tail():
        pltpu.sync_copy(zrow, acc.at[pl.ds(rows_per_sub * _NS, n_tail)])

    plsc.subcore_barrier()

    def start_meta(k, hb, tb, yb, ib, sem):
        off = eb + jnp.minimum(k, n_chunks - 1) * _B
        pltpu.make_async_copy(head_ref.at[pl.ds(off, _B)], hb, sem).start()
        pltpu.make_async_copy(tail_ref.at[pl.ds(off, _B)], tb, sem).start()
        pltpu.make_async_copy(type_ref.at[pl.ds(off, _B)], yb, sem).start()
        pltpu.make_async_copy(imp_ref.at[pl.ds(off, _B)], ib, sem).start()

    def wait_meta(hb, tb, yb, ib, sem):
        pltpu.make_async_copy(head_ref.at[pl.ds(0, _B)], hb, sem).wait()
        pltpu.make_async_copy(tail_ref.at[pl.ds(0, _B)], tb, sem).wait()
        pltpu.make_async_copy(type_ref.at[pl.ds(0, _B)], yb, sem).wait()
        pltpu.make_async_copy(imp_ref.at[pl.ds(0, _B)], ib, sem).wait()

    def body(hb, tb, yb, ib):
        cp_r = pltpu.make_async_copy(ent_ref.at[tb], rows, sem_g)
        cp_r.start()
        cp_y = pltpu.make_async_copy(rel_ref.at[yb], relrows, sem_g)
        cp_y.start()

        for g in range(_B // 16):
            sl = pl.ds(g * 16, 16)
            aidx[sl] = hb[sl] * n_rel + yb[sl]

        cp_a = pltpu.make_async_copy(att_ref.at[aidx], abuf, sem_g)
        cp_a.start()
        cp_a.wait(); cp_y.wait(); cp_r.wait()

        def grp(g, cy):
            sl = pl.ds(g * 16, 16)
            cfv = abuf[sl] * ib[sl]
            base = g * 16
            for j2 in range(16):
                cf = cfv[j2]
                for db in range(d // 16):
                    dsl = pl.ds(db * 16, 16)
                    rows[base + j2, dsl] = (rows[base + j2, dsl]
                                            * relrows[base + j2, dsl] * cf)
            return cy

        lax.fori_loop(0, _B // 16, grp, 0)

        pltpu.sync_copy(rows, acc.at[hb], add=True)

    start_meta(0, hb0, tb0, yb0, ib0, sem_m0)

    def step(k, carry):
        e_ch = 2 * k
        wait_meta(hb0, tb0, yb0, ib0, sem_m0)
        start_meta(e_ch + 1, hb1, tb1, yb1, ib1, sem_m1)
        body(hb0, tb0, yb0, ib0)
        wait_meta(hb1, tb1, yb1, ib1, sem_m1)
        start_meta(e_ch + 2, hb0, tb0, yb0, ib0, sem_m0)
        body(hb1, tb1, yb1, ib1)
        return carry

    lax.fori_loop(0, (n_chunks - 1) // 2, step, 0)

    wait_meta(hb0, tb0, yb0, ib0, sem_m0)
    body(hb0, tb0, yb0, ib0)

    plsc.subcore_barrier()

    pltpu.sync_copy(acc.at[pl.ds(rb, rows_per_sub)],
                    out_ref.at[c, pl.ds(rb, rows_per_sub)])

    @pl.when(s == 0)
    def _copy_tail():
        pltpu.sync_copy(acc.at[pl.ds(rows_per_sub * _NS, n_tail)],
                        out_ref.at[c, pl.ds(rows_per_sub * _NS, n_tail)])


def _edge_path(att_flat, entity_emb, rel2d, head, tail, etype, imp):
    n_ent, d = entity_emb.shape
    n_rel = rel2d.shape[0]
    mesh = plsc.VectorSubcoreMesh(core_axis_name="c", subcore_axis_name="s")
    k = functools.partial(
        pl.kernel,
        out_type=jax.ShapeDtypeStruct((_NC, n_ent, d), jnp.float32),
        mesh=mesh,
        scratch_types=[
            pltpu.VMEM((_B,), jnp.int32),
            pltpu.VMEM((_B,), jnp.int32),
            pltpu.VMEM((_B,), jnp.int32),
            pltpu.VMEM((_B,), jnp.float32),
            pltpu.VMEM((_B,), jnp.int32),
            pltpu.VMEM((_B,), jnp.int32),
            pltpu.VMEM((_B,), jnp.int32),
            pltpu.VMEM((_B,), jnp.float32),
            pltpu.VMEM((_B,), jnp.float32),
            pltpu.VMEM((_B,), jnp.int32),
            pltpu.VMEM((_B, d), jnp.float32),
            pltpu.VMEM((_B, d), jnp.float32),
            pltpu.VMEM((16, d), jnp.float32),
            pltpu.VMEM_SHARED((n_ent, d), jnp.float32),
            pltpu.SemaphoreType.DMA,
            pltpu.SemaphoreType.DMA,
            pltpu.SemaphoreType.DMA,
        ],
    )(_edge_body)
    return k(att_flat, entity_emb, rel2d, head, tail, etype, imp)


def _combine_body(p_ref, o_ref):
    o_ref[...] = p_ref[0] + p_ref[1]


def _combine(partials):
    _, n_ent, d = partials.shape
    blk = 2000
    return pl.pallas_call(
        _combine_body,
        grid=(n_ent // blk,),
        in_specs=[pl.BlockSpec((_NC, blk, d), lambda i: (0, i, 0))],
        out_specs=pl.BlockSpec((blk, d), lambda i: (i, 0)),
        out_shape=jax.ShapeDtypeStruct((n_ent, d), jnp.float32),
    )(partials)


def _user_body(inter_ref, ent_ref, icm_ref, item_ref, rel_ref, usr_ref,
               clsw_ref, out_ref):
    ua = jnp.dot(inter_ref[...], ent_ref[...], preferred_element_type=jnp.float32)
    z = jnp.dot(usr_ref[...], clsw_ref[...].T, preferred_element_type=jnp.float32)
    z = z - jnp.max(z, axis=1, keepdims=True)
    ez = jnp.exp(z)
    catt = ez / jnp.sum(ez, axis=1, keepdims=True)
    item2 = item_ref[...] * jnp.sum(rel_ref[...], axis=0, keepdims=True)
    n_cls = clsw_ref.shape[0]
    for cc in range(n_cls):
        dw = jnp.dot(icm_ref[cc], item2, preferred_element_type=jnp.float32)
        ua = ua + catt[:, cc:cc + 1] * dw
    out_ref[...] = ua


def _user_path(interact_mat, entity_emb, inter_cls_mat, item_emb,
               relation_emb, user_emb, usr_cls_w):
    n_usr, n_ent = interact_mat.shape
    d = entity_emb.shape[1]
    n_cls, _, n_itm = inter_cls_mat.shape
    ub = 128
    grid = (n_usr // ub,)
    return pl.pallas_call(
        _user_body,
        grid=grid,
        in_specs=[
            pl.BlockSpec((ub, n_ent), lambda i: (i, 0)),
            pl.BlockSpec((n_ent, d), lambda i: (0, 0)),
            pl.BlockSpec((n_cls, ub, n_itm), lambda i: (0, i, 0)),
            pl.BlockSpec((n_itm, d), lambda i: (0, 0)),
            pl.BlockSpec((relation_emb.shape[0], d), lambda i: (0, 0)),
            pl.BlockSpec((ub, d), lambda i: (i, 0)),
            pl.BlockSpec((n_cls, d), lambda i: (0, 0)),
        ],
        out_specs=pl.BlockSpec((ub, d), lambda i: (i, 0)),
        out_shape=jax.ShapeDtypeStruct((n_usr, d), jnp.float32),
    )(interact_mat, entity_emb, inter_cls_mat, item_emb, relation_emb,
      user_emb, usr_cls_w)


def kernel(entity_emb, item_emb, user_emb, latent_emb, relation_emb,
           edge_index, edge_type, edge_imp, interact_mat, disen_weight_att,
           ent_rel_w, usr_cls_w, inter_cls_mat):
    att = _entity_relation_att(entity_emb, relation_emb)
    att_flat = att.reshape(-1)

    partials = _edge_path(att_flat, entity_emb, relation_emb,
                          edge_index[0], edge_index[1],
                          edge_type, edge_imp)
    entity_agg = _combine(partials)

    user_agg = _user_path(interact_mat, entity_emb, inter_cls_mat, item_emb,
                          relation_emb, user_emb, usr_cls_w)
    return (entity_agg, user_agg)


# speedup vs baseline: 1.3750x; 1.3750x over previous
import functools

import jax
import jax.numpy as jnp
from jax import lax
from jax.experimental import pallas as pl
from jax.experimental.pallas import tpu as pltpu
from jax.experimental.pallas import tpu_sc as plsc

_NC = 2
_NS = 16
_B = 80


def _att_body(ent_ref, rel_ref, out_ref):
    z = jnp.dot(ent_ref[...], rel_ref[...].T, preferred_element_type=jnp.float32)
    z = z - jnp.max(z, axis=1, keepdims=True)
    e = jnp.exp(z)
    out_ref[...] = e / jnp.sum(e, axis=1, keepdims=True)


def _entity_relation_att(entity_emb, relation_emb):
    n_ent, _ = entity_emb.shape
    n_rel = relation_emb.shape[0]
    return pl.pallas_call(
        _att_body,
        out_shape=jax.ShapeDtypeStruct((n_ent, n_rel), jnp.float32),
    )(entity_emb, relation_emb)


def _edge_body(att_ref, ent_ref, rel_ref, head_ref, tail_ref, type_ref,
               imp_ref, out_ref,
               hb0, tb0, yb0, ib0, hb1, tb1, yb1, ib1,
               ab0, ai0, rw0, ab1, ai1, rw1, relv, zrow, acc,
               sem_m0, sem_m1, sem_g0, sem_g1):
    c = lax.axis_index("c")
    s = lax.axis_index("s")
    n_ent, d = acc.shape
    n_rel = relv.shape[0] // d
    e_total = head_ref.shape[0]
    e_per_w = e_total // (_NC * _NS)
    n_chunks = e_per_w // _B
    rows_per_sub = 624
    n_tail = n_ent - rows_per_sub * _NS
    rb = s * rows_per_sub
    wid = c * _NS + s
    eb = wid * e_per_w
    iota16 = lax.iota(jnp.int32, 16)
    z16 = jnp.zeros((16,), jnp.float32)

    pltpu.sync_copy(rel_ref, relv)

    nz = zrow.shape[0]

    @plsc.parallel_loop(0, nz)
    def _zero_rows(i):
        for k in range(d // 16):
            zrow[i, pl.ds(k * 16, 16)] = z16

    @plsc.parallel_loop(0, rows_per_sub // nz)
    def _zero_acc(j):
        pltpu.sync_copy(zrow, acc.at[pl.ds(rb + j * nz, nz)])

    @pl.when(s == 0)
    def _zero_tail():
        pltpu.sync_copy(zrow, acc.at[pl.ds(rows_per_sub * _NS, n_tail)])

    plsc.subcore_barrier()

    def start_meta(k, hb, tb, yb, ib, sem):
        off = eb + jnp.minimum(k, n_chunks - 1) * _B
        pltpu.make_async_copy(head_ref.at[pl.ds(off, _B)], hb, sem).start()
        pltpu.make_async_copy(tail_ref.at[pl.ds(off, _B)], tb, sem).start()
        pltpu.make_async_copy(type_ref.at[pl.ds(off, _B)], yb, sem).start()
        pltpu.make_async_copy(imp_ref.at[pl.ds(off, _B)], ib, sem).start()

    def wait_meta(hb, tb, yb, ib, sem):
        pltpu.make_async_copy(head_ref.at[pl.ds(0, _B)], hb, sem).wait()
        pltpu.make_async_copy(tail_ref.at[pl.ds(0, _B)], tb, sem).wait()
        pltpu.make_async_copy(type_ref.at[pl.ds(0, _B)], yb, sem).wait()
        pltpu.make_async_copy(imp_ref.at[pl.ds(0, _B)], ib, sem).wait()

    def start_gather(hb, tb, yb, rw, ai, ab, sem):
        pltpu.make_async_copy(ent_ref.at[tb], rw, sem).start()
        for g in range(_B // 16):
            sl = pl.ds(g * 16, 16)
            ai[sl] = hb[sl] * n_rel + yb[sl]
        pltpu.make_async_copy(att_ref.at[ai], ab, sem).start()

    def finish(hb, tb, yb, ib, rw, ai, ab, sem):
        pltpu.make_async_copy(att_ref.at[ai], ab, sem).wait()
        pltpu.make_async_copy(ent_ref.at[tb], rw, sem).wait()

        def grp(g, cy):
            sl = pl.ds(g * 16, 16)
            tyv = yb[sl] * d
            cfv = ab[sl] * ib[sl]
            base = g * 16
            for j2 in range(16):
                rbase = tyv[j2]
                cf = cfv[j2]
                for db in range(d // 16):
                    dsl = pl.ds(db * 16, 16)
                    rv = relv[pl.ds(rbase + db * 16, 16)]
                    rw[base + j2, dsl] = rw[base + j2, dsl] * rv * cf
            return cy

        lax.fori_loop(0, _B // 16, grp, 0)

        pltpu.sync_copy(rw, acc.at[hb], add=True)

    start_meta(0, hb0, tb0, yb0, ib0, sem_m0)
    wait_meta(hb0, tb0, yb0, ib0, sem_m0)
    start_meta(1, hb1, tb1, yb1, ib1, sem_m1)
    start_gather(hb0, tb0, yb0, rw0, ai0, ab0, sem_g0)

    def step(k, carry):
        e_ch = 2 * k
        wait_meta(hb1, tb1, yb1, ib1, sem_m1)
        start_gather(hb1, tb1, yb1, rw1, ai1, ab1, sem_g1)
        finish(hb0, tb0, yb0, ib0, rw0, ai0, ab0, sem_g0)
        start_meta(e_ch + 2, hb0, tb0, yb0, ib0, sem_m0)
        wait_meta(hb0, tb0, yb0, ib0, sem_m0)
        start_gather(hb0, tb0, yb0, rw0, ai0, ab0, sem_g0)
        finish(hb1, tb1, yb1, ib1, rw1, ai1, ab1, sem_g1)
        start_meta(e_ch + 3, hb1, tb1, yb1, ib1, sem_m1)
        return carry

    lax.fori_loop(0, (n_chunks - 1) // 2, step, 0)

    wait_meta(hb1, tb1, yb1, ib1, sem_m1)
    finish(hb0, tb0, yb0, ib0, rw0, ai0, ab0, sem_g0)

    plsc.subcore_barrier()

    pltpu.sync_copy(acc.at[pl.ds(rb, rows_per_sub)],
                    out_ref.at[c, pl.ds(rb, rows_per_sub)])

    @pl.when(s == 0)
    def _copy_tail():
        pltpu.sync_copy(acc.at[pl.ds(rows_per_sub * _NS, n_tail)],
                        out_ref.at[c, pl.ds(rows_per_sub * _NS, n_tail)])


def _edge_path(att_flat, entity_emb, rel_flat, n_rel, head, tail, etype, imp):
    n_ent, d = entity_emb.shape
    mesh = plsc.VectorSubcoreMesh(core_axis_name="c", subcore_axis_name="s")
    k = functools.partial(
        pl.kernel,
        out_type=jax.ShapeDtypeStruct((_NC, n_ent, d), jnp.float32),
        mesh=mesh,
        scratch_types=[
            pltpu.VMEM((_B,), jnp.int32),
            pltpu.VMEM((_B,), jnp.int32),
            pltpu.VMEM((_B,), jnp.int32),
            pltpu.VMEM((_B,), jnp.float32),
            pltpu.VMEM((_B,), jnp.int32),
            pltpu.VMEM((_B,), jnp.int32),
            pltpu.VMEM((_B,), jnp.int32),
            pltpu.VMEM((_B,), jnp.float32),
            pltpu.VMEM((_B,), jnp.float32),
            pltpu.VMEM((_B,), jnp.int32),
            pltpu.VMEM((_B, d), jnp.float32),
            pltpu.VMEM((_B,), jnp.float32),
            pltpu.VMEM((_B,), jnp.int32),
            pltpu.VMEM((_B, d), jnp.float32),
            pltpu.VMEM((n_rel * d,), jnp.float32),
            pltpu.VMEM((16, d), jnp.float32),
            pltpu.VMEM_SHARED((n_ent, d), jnp.float32),
            pltpu.SemaphoreType.DMA,
            pltpu.SemaphoreType.DMA,
            pltpu.SemaphoreType.DMA,
            pltpu.SemaphoreType.DMA,
        ],
    )(_edge_body)
    return k(att_flat, entity_emb, rel_flat, head, tail, etype, imp)


def _combine_body(p_ref, o_ref):
    o_ref[...] = p_ref[0] + p_ref[1]


def _combine(partials):
    _, n_ent, d = partials.shape
    blk = 2000
    return pl.pallas_call(
        _combine_body,
        grid=(n_ent // blk,),
        in_specs=[pl.BlockSpec((_NC, blk, d), lambda i: (0, i, 0))],
        out_specs=pl.BlockSpec((blk, d), lambda i: (i, 0)),
        out_shape=jax.ShapeDtypeStruct((n_ent, d), jnp.float32),
    )(partials)


def _user_body(inter_ref, ent_ref, icm_ref, item_ref, rel_ref, usr_ref,
               clsw_ref, out_ref):
    ua = jnp.dot(inter_ref[...], ent_ref[...], preferred_element_type=jnp.float32)
    z = jnp.dot(usr_ref[...], clsw_ref[...].T, preferred_element_type=jnp.float32)
    z = z - jnp.max(z, axis=1, keepdims=True)
    ez = jnp.exp(z)
    catt = ez / jnp.sum(ez, axis=1, keepdims=True)
    item2 = item_ref[...] * jnp.sum(rel_ref[...], axis=0, keepdims=True)
    n_cls = clsw_ref.shape[0]
    for cc in range(n_cls):
        dw = jnp.dot(icm_ref[cc], item2, preferred_element_type=jnp.float32)
        ua = ua + catt[:, cc:cc + 1] * dw
    out_ref[...] = ua


def _user_path(interact_mat, entity_emb, inter_cls_mat, item_emb,
               relation_emb, user_emb, usr_cls_w):
    n_usr, n_ent = interact_mat.shape
    d = entity_emb.shape[1]
    n_cls, _, n_itm = inter_cls_mat.shape
    ub = 128
    grid = (n_usr // ub,)
    return pl.pallas_call(
        _user_body,
        grid=grid,
        in_specs=[
            pl.BlockSpec((ub, n_ent), lambda i: (i, 0)),
            pl.BlockSpec((n_ent, d), lambda i: (0, 0)),
            pl.BlockSpec((n_cls, ub, n_itm), lambda i: (0, i, 0)),
            pl.BlockSpec((n_itm, d), lambda i: (0, 0)),
            pl.BlockSpec((relation_emb.shape[0], d), lambda i: (0, 0)),
            pl.BlockSpec((ub, d), lambda i: (i, 0)),
            pl.BlockSpec((n_cls, d), lambda i: (0, 0)),
        ],
        out_specs=pl.BlockSpec((ub, d), lambda i: (i, 0)),
        out_shape=jax.ShapeDtypeStruct((n_usr, d), jnp.float32),
    )(interact_mat, entity_emb, inter_cls_mat, item_emb, relation_emb,
      user_emb, usr_cls_w)


def kernel(entity_emb, item_emb, user_emb, latent_emb, relation_emb,
           edge_index, edge_type, edge_imp, interact_mat, disen_weight_att,
           ent_rel_w, usr_cls_w, inter_cls_mat):
    att = _entity_relation_att(entity_emb, relation_emb)
    att_flat = att.reshape(-1)

    partials = _edge_path(att_flat, entity_emb, relation_emb.reshape(-1),
                          relation_emb.shape[0], edge_index[0], edge_index[1],
                          edge_type, edge_imp)
    entity_agg = _combine(partials)

    user_agg = _user_path(interact_mat, entity_emb, inter_cls_mat, item_emb,
                          relation_emb, user_emb, usr_cls_w)
    return (entity_agg, user_agg)
